# baseline (speedup 1.0000x reference)
import functools

import jax
import jax.numpy as jnp
from jax import lax
from jax.experimental import pallas as pl
from jax.experimental.pallas import tpu as pltpu
from jax.experimental.pallas import tpu_sc as plsc

N = 10000
NP = 10240
C = 256
B = 64
E = 160000
ETOT = E + N
NSUB = 16
K = 128
CH = 88
EP = NSUB * CH * K
NPS = NP // NSUB
RB = 1024
GRID = NP // RB

_F32 = jnp.float32
_HI = lax.Precision.HIGHEST


def _dot(a, b):
    return lax.dot_general(a, b, (((1,), (0,)), ((), ())),
                           precision=_HI, preferred_element_type=_F32)


def _tc_layer_body(x_ref, w_ref, a_ref, h0_ref, h1_ref, as_ref, ad_ref, m_ref):
    i = pl.program_id(0)
    h = _dot(x_ref[...], w_ref[...])
    h0_ref[...] = h[:, :128]
    h1_ref[...] = h[:, 128:]
    aa = _dot(h, a_ref[...])
    as_ref[...] = aa[:, 0:1]
    ad_ref[...] = aa[:, 1:2]
    m2 = jnp.max(aa, axis=0, keepdims=True)
    m2p = jnp.pad(m2, ((0, 0), (0, 126)), constant_values=-1e30)

    @pl.when(i == 0)
    def _():
        m_ref[...] = jnp.full((1, 128), -1e30, _F32)

    m_ref[...] = jnp.maximum(m_ref[...], m2p)


def _tc_layer0(xp, wp, a2):
    fin = xp.shape[1]
    return pl.pallas_call(
        _tc_layer_body,
        grid=(GRID,),
        in_specs=[
            pl.BlockSpec((RB, fin), lambda i: (i, 0)),
            pl.BlockSpec((fin, C), lambda i: (0, 0)),
            pl.BlockSpec((C, 2), lambda i: (0, 0)),
        ],
        out_specs=[
            pl.BlockSpec((RB, 128), lambda i: (i, 0)),
            pl.BlockSpec((RB, 128), lambda i: (i, 0)),
            pl.BlockSpec((RB, 1), lambda i: (i, 0)),
            pl.BlockSpec((RB, 1), lambda i: (i, 0)),
            pl.BlockSpec((1, 128), lambda i: (0, 0)),
        ],
        out_shape=[
            jax.ShapeDtypeStruct((NP, 128), _F32),
            jax.ShapeDtypeStruct((NP, 128), _F32),
            jax.ShapeDtypeStruct((NP, 1), _F32),
            jax.ShapeDtypeStruct((NP, 1), _F32),
            jax.ShapeDtypeStruct((1, 128), _F32),
        ],
    )(xp, wp, a2)


def _tc_layer12_body(r0_ref, r1_ref, d_ref, b_ref, w_ref, a_ref,
                     h0_ref, h1_ref, as_ref, ad_ref, m_ref):
    i = pl.program_id(0)
    xr = jnp.concatenate([r0_ref[...], r1_ref[...]], axis=1)
    xr = xr / jnp.maximum(d_ref[...], 1e-16) + b_ref[...]
    xr = jnp.maximum(xr, 0.0)
    h = _dot(xr, w_ref[...])
    h0_ref[...] = h[:, :128]
    h1_ref[...] = h[:, 128:]
    aa = _dot(h, a_ref[...])
    as_ref[...] = aa[:, 0:1]
    ad_ref[...] = aa[:, 1:2]
    m2 = jnp.max(aa, axis=0, keepdims=True)
    m2p = jnp.pad(m2, ((0, 0), (0, 126)), constant_values=-1e30)

    @pl.when(i == 0)
    def _():
        m_ref[...] = jnp.full((1, 128), -1e30, _F32)

    m_ref[...] = jnp.maximum(m_ref[...], m2p)


def _tc_layer12(raw0, raw1, den2, b2, w, a2):
    return pl.pallas_call(
        _tc_layer12_body,
        grid=(GRID,),
        in_specs=[
            pl.BlockSpec((RB, 128), lambda i: (i, 0)),
            pl.BlockSpec((RB, 128), lambda i: (i, 0)),
            pl.BlockSpec((RB, 1), lambda i: (i, 0)),
            pl.BlockSpec((1, C), lambda i: (0, 0)),
            pl.BlockSpec((C, C), lambda i: (0, 0)),
            pl.BlockSpec((C, 2), lambda i: (0, 0)),
        ],
        out_specs=[
            pl.BlockSpec((RB, 128), lambda i: (i, 0)),
            pl.BlockSpec((RB, 128), lambda i: (i, 0)),
            pl.BlockSpec((RB, 1), lambda i: (i, 0)),
            pl.BlockSpec((RB, 1), lambda i: (i, 0)),
            pl.BlockSpec((1, 128), lambda i: (0, 0)),
        ],
        out_shape=[
            jax.ShapeDtypeStruct((NP, 128), _F32),
            jax.ShapeDtypeStruct((NP, 128), _F32),
            jax.ShapeDtypeStruct((NP, 1), _F32),
            jax.ShapeDtypeStruct((NP, 1), _F32),
            jax.ShapeDtypeStruct((1, 128), _F32),
        ],
    )(raw0, raw1, den2, b2, w, a2)


def _head_body(r0_ref, r1_ref, d_ref, b_ref, bat_ref, lw_ref, lb_ref,
               o_ref, s_ref, c_ref):
    i = pl.program_id(0)

    @pl.when(i == 0)
    def _():
        s_ref[...] = jnp.zeros((B, C), _F32)
        c_ref[...] = jnp.zeros((B, 128), _F32)

    h3 = jnp.concatenate([r0_ref[...], r1_ref[...]], axis=1)
    h3 = h3 / jnp.maximum(d_ref[...], 1e-16) + b_ref[...]
    bat = bat_ref[0, 0, :]
    rid = lax.broadcasted_iota(jnp.int32, (B, RB), 0)
    oh = (bat[None, :] == rid).astype(_F32)
    s_ref[...] += _dot(oh, h3)
    cnt = jnp.sum(oh, axis=1, keepdims=True)
    c_ref[...] += jnp.broadcast_to(cnt, (B, 128))

    @pl.when(i == GRID - 1)
    def _():
        pooled = s_ref[...] / jnp.maximum(c_ref[:, 0:1], 1.0)
        val = _dot(pooled, lw_ref[...]) + lb_ref[...]
        o_ref[...] = jax.nn.sigmoid(val)


def _head(raw0, raw1, den2, b2, bat3, lw, lb2):
    return pl.pallas_call(
        _head_body,
        grid=(GRID,),
        in_specs=[
            pl.BlockSpec((RB, 128), lambda i: (i, 0)),
            pl.BlockSpec((RB, 128), lambda i: (i, 0)),
            pl.BlockSpec((RB, 1), lambda i: (i, 0)),
            pl.BlockSpec((1, C), lambda i: (0, 0)),
            pl.BlockSpec((1, 1, RB), lambda i: (i, 0, 0)),
            pl.BlockSpec((C, 1), lambda i: (0, 0)),
            pl.BlockSpec((1, 1), lambda i: (0, 0)),
        ],
        out_specs=pl.BlockSpec((B, 1), lambda i: (0, 0)),
        out_shape=jax.ShapeDtypeStruct((B, 1), _F32),
        scratch_shapes=[
            pltpu.VMEM((B, C), _F32),
            pltpu.VMEM((B, 128), _F32),
        ],
    )(raw0, raw1, den2, b2, bat3, lw, lb2)


_MESH = plsc.VectorSubcoreMesh(core_axis_name="c", subcore_axis_name="s")

_SC_CP = pltpu.CompilerParams()
if "needs_layout_passes" in pltpu.CompilerParams.__dataclass_fields__:
    import dataclasses as _dc
    _SC_CP = _dc.replace(_SC_CP, needs_layout_passes=False)

NBLK = CH // 8


@functools.partial(
    pl.kernel,
    mesh=_MESH,
    compiler_params=_SC_CP,
    out_type=[
        jax.ShapeDtypeStruct((NSUB, CH, K), _F32),
        jax.ShapeDtypeStruct((NP,), _F32),
    ],
    scratch_types=[
        pltpu.VMEM((NP,), _F32),
        pltpu.VMEM((NP,), _F32),
        pltpu.VMEM((CH, K), jnp.int32),
        pltpu.VMEM((CH, K), jnp.int32),
        pltpu.VMEM((CH, K), _F32),
        pltpu.VMEM((2, K), _F32),
        pltpu.VMEM((16,), _F32),
        pltpu.VMEM_SHARED((NP,), _F32),
        pltpu.SemaphoreType.DMA((2,)),
    ],
)
def _sc_ee(srcm_h, dstm_h, as_h, ad_h, m_h, zd_h, eem_h, den_h,
           asb, adb, srcb, dstb, eeball, eeb, mb, denacc, dsem):
    c = lax.axis_index("c")
    s = lax.axis_index("s")
    nb = s * NPS

    @pl.when(c == 0)
    def _():
        pltpu.sync_copy(as_h, asb)
        pltpu.sync_copy(ad_h, adb)
        pltpu.sync_copy(m_h, mb)
        pltpu.sync_copy(srcm_h.at[s], srcb)
        pltpu.sync_copy(dstm_h.at[s], dstb)
        pltpu.sync_copy(zd_h, denacc.at[pl.ds(nb, NPS)])
        plsc.subcore_barrier()

        mv = mb[...]
        ebase = s * (CH * K)
        iota16 = lax.broadcasted_iota(jnp.int32, (16,), 0)

        def wait_den(bb, idx):
            pltpu.make_async_copy(eeb.at[bb], denacc.at[idx], dsem.at[bb]).wait()

        @pl.loop(0, CH // 2)
        def _(ii):
            for b in range(2):
                ci = ii * 2 + b
                idx_d = dstb.at[ci]
                pl.when(ii > 0)(lambda: wait_den(b, idx_d))
                for g in range(K // 16):
                    sv = srcb[ci, pl.ds(g * 16, 16)]
                    dv = dstb[ci, pl.ds(g * 16, 16)]
                    a1 = plsc.load_gather(asb, [sv])
                    a2 = plsc.load_gather(adb, [dv])
                    z = a1 + a2
                    e = jnp.maximum(z, 0.2 * z)
                    ee = jnp.exp(e - mv)
                    gid = ebase + ci * K + g * 16 + iota16
                    ee = jnp.where(gid < ETOT, ee, 0.0)
                    eeball[ci, pl.ds(g * 16, 16)] = ee
                    eeb[b, pl.ds(g * 16, 16)] = ee
                pltpu.async_copy(eeb.at[b], denacc.at[idx_d],
                                 dsem.at[b], add=True)

        wait_den(0, dstb.at[0])
        wait_den(1, dstb.at[1])
        pltpu.sync_copy(eeball, eem_h.at[s])
        plsc.subcore_barrier()
        pltpu.sync_copy(denacc.at[pl.ds(nb, NPS)], den_h.at[pl.ds(nb, NPS)])


@functools.partial(
    pl.kernel,
    mesh=_MESH,
    compiler_params=_SC_CP,
    out_type=[
        jax.ShapeDtypeStruct((NP, 128), _F32),
        jax.ShapeDtypeStruct((NP, 128), _F32),
    ],
    scratch_types=[
        pltpu.VMEM((2, 8, K), jnp.int32),
        pltpu.VMEM((2, 8, K), jnp.int32),
        pltpu.VMEM((2, 8, K), _F32),
        pltpu.VMEM((2, K, 128), _F32),
        pltpu.VMEM_SHARED((NP, 128), _F32),
        pltpu.SemaphoreType.DMA((2,)),
        pltpu.SemaphoreType.DMA((2,)),
        pltpu.SemaphoreType.DMA((2,)),
    ],
)
def _sc_agg(srcm_h, dstm_h, eem_h, hh0_h, hh1_h, zr_h, raw0_h, raw1_h,
            srcst, dstst, eest, rows, accum, grsem, ssem, stsem):
    c = lax.axis_index("c")
    s = lax.axis_index("s")
    nb = s * NPS

    pltpu.sync_copy(zr_h, accum.at[pl.ds(nb, NPS)])
    pltpu.sync_copy(srcm_h.at[s, pl.ds(0, 8)], srcst.at[0])
    pltpu.sync_copy(dstm_h.at[s, pl.ds(0, 8)], dstst.at[0])
    pltpu.sync_copy(eem_h.at[s, pl.ds(0, 8)], eest.at[0])
    plsc.subcore_barrier()

    z16 = jnp.zeros((16,), jnp.int32)

    def _half(hh_h):
        def g_issue(bb, idx):
            pltpu.async_copy(hh_h.at[idx], rows.at[bb], grsem.at[bb])

        def g_wait(bb, idx):
            pltpu.make_async_copy(hh_h.at[idx], rows.at[bb],
                                  grsem.at[bb]).wait()

        def s_wait(bb, idx):
            pltpu.make_async_copy(rows.at[bb], accum.at[idx],
                                  ssem.at[bb]).wait()

        g_issue(0, srcst.at[0, 0])

        @pl.loop(0, NBLK)
        def _(blk):
            p = blk % 2
            for j in range(8):
                b = j % 2
                idx_s = srcst.at[p, j]
                idx_d = dstst.at[p, j]

                g_wait(b, idx_s)

                if j == 0:
                    pl.when(blk > 0)(lambda: s_wait(1 - b, idx_d))
                    g_issue(1, srcst.at[p, 1])
                elif j < 7:
                    s_wait(1 - b, idx_d)
                    g_issue(1 - b, srcst.at[p, j + 1])
                else:
                    def tail():
                        s_wait(1 - b, idx_d)
                        for sth, stv in ((srcm_h, srcst), (dstm_h, dstst),
                                         (eem_h, eest)):
                            pltpu.make_async_copy(
                                sth.at[s, pl.ds((blk + 1) * 8, 8)],
                                stv.at[1 - p], stsem.at[1 - p]).wait()
                        g_issue(1 - b, srcst.at[1 - p, 0])
                    pl.when(blk < NBLK - 1)(tail)

                if j == 2:
                    def refill():
                        for sth, stv in ((srcm_h, srcst), (dstm_h, dstst),
                                         (eem_h, eest)):
                            pltpu.async_copy(
                                sth.at[s, pl.ds((blk + 1) * 8, 8)],
                                stv.at[1 - p], stsem.at[1 - p])
                    pl.when(blk < NBLK - 1)(refill)

                @plsc.parallel_loop(0, K, unroll=8)
                def _(k):
                    sc16 = plsc.load_gather(
                        eest, [z16 + p, z16 + j, z16 + k])
                    for jj in range(8):
                        sl = pl.ds(jj * 16, 16)
                        rows[b, k, sl] = rows[b, k, sl] * sc16

                pltpu.async_copy(rows.at[b], accum.at[idx_d],
                                 ssem.at[b], add=True)

        s_wait(0, dstst.at[0, 0])
        s_wait(1, dstst.at[0, 0])

    @pl.when(c == 0)
    def _():
        _half(hh0_h)

    @pl.when(c == 1)
    def _():
        _half(hh1_h)

    plsc.subcore_barrier()

    @pl.when(c == 0)
    def _():
        pltpu.sync_copy(accum.at[pl.ds(nb, NPS)], raw0_h.at[pl.ds(nb, NPS)])

    @pl.when(c == 1)
    def _():
        pltpu.sync_copy(accum.at[pl.ds(nb, NPS)], raw1_h.at[pl.ds(nb, NPS)])


def kernel(x, edge_index, batch, W0, att_src0, att_dst0, bias0,
           W1, att_src1, att_dst1, bias1, W2, att_src2, att_dst2, bias2,
           lin_W, lin_b):
    f32 = _F32
    xp = jnp.pad(x, ((0, NP - N), (0, 3)))
    w0p = jnp.pad(W0, ((0, 3), (0, 0)))
    a0 = jnp.stack([att_src0, att_dst0], axis=1)
    a1 = jnp.stack([att_src1, att_dst1], axis=1)
    a2 = jnp.stack([att_src2, att_dst2], axis=1)

    loop = jnp.arange(N, dtype=jnp.int32)
    padi = jnp.arange(EP - ETOT, dtype=jnp.int32) % N
    srcm = jnp.concatenate([edge_index[0], loop, padi]).reshape(NSUB, CH, K)
    dstm = jnp.concatenate([edge_index[1], loop, padi]).reshape(NSUB, CH, K)

    zr = jnp.zeros((NPS, 128), f32)
    zd = jnp.zeros((NPS,), f32)
    batp = jnp.concatenate(
        [batch, jnp.full((NP - N,), B, jnp.int32)]).reshape(GRID, 1, RB)

    def msplat(mx):
        m = mx[0, 0] + mx[0, 1]
        m = jnp.maximum(m, 0.2 * m)
        return jnp.full((16,), m, f32)

    hh0, hh1, asv, adv, mx = _tc_layer0(xp, w0p, a0)
    eem, den = _sc_ee(srcm, dstm, asv.reshape(NP), adv.reshape(NP),
                      msplat(mx), zd)
    raw0, raw1 = _sc_agg(srcm, dstm, eem, hh0, hh1, zr)

    hh0, hh1, asv, adv, mx = _tc_layer12(
        raw0, raw1, den.reshape(NP, 1), bias0.reshape(1, C), W1, a1)
    eem, den = _sc_ee(srcm, dstm, asv.reshape(NP), adv.reshape(NP),
                      msplat(mx), zd)
    raw0, raw1 = _sc_agg(srcm, dstm, eem, hh0, hh1, zr)

    hh0, hh1, asv, adv, mx = _tc_layer12(
        raw0, raw1, den.reshape(NP, 1), bias1.reshape(1, C), W2, a2)
    eem, den = _sc_ee(srcm, dstm, asv.reshape(NP), adv.reshape(NP),
                      msplat(mx), zd)
    raw0, raw1 = _sc_agg(srcm, dstm, eem, hh0, hh1, zr)

    return _head(raw0, raw1, den.reshape(NP, 1), bias2.reshape(1, C),
                 batp, lin_W, lin_b.reshape(1, 1))


# speedup vs baseline: 1.0166x; 1.0166x over previous
import functools

import jax
import jax.numpy as jnp
from jax import lax
from jax.experimental import pallas as pl
from jax.experimental.pallas import tpu as pltpu
from jax.experimental.pallas import tpu_sc as plsc

N = 10000
NP = 10240
C = 256
B = 64
E = 160000
ETOT = E + N
NSUB = 16
K = 128
CH = 88
EP = NSUB * CH * K
NPS = NP // NSUB
RB = 1024
GRID = NP // RB

_F32 = jnp.float32
_HI = lax.Precision.HIGHEST


def _dot(a, b):
    return lax.dot_general(a, b, (((1,), (0,)), ((), ())),
                           precision=_HI, preferred_element_type=_F32)


def _tc_layer_body(x_ref, w_ref, a_ref, h0_ref, h1_ref, as_ref, ad_ref, m_ref):
    i = pl.program_id(0)
    h = _dot(x_ref[...], w_ref[...])
    h0_ref[...] = h[:, :128]
    h1_ref[...] = h[:, 128:]
    aa = _dot(h, a_ref[...])
    as_ref[...] = aa[:, 0:1]
    ad_ref[...] = aa[:, 1:2]
    m2 = jnp.max(aa, axis=0, keepdims=True)
    m2p = jnp.pad(m2, ((0, 0), (0, 126)), constant_values=-1e30)

    @pl.when(i == 0)
    def _():
        m_ref[...] = jnp.full((1, 128), -1e30, _F32)

    m_ref[...] = jnp.maximum(m_ref[...], m2p)


def _tc_layer0(xp, wp, a2):
    fin = xp.shape[1]
    return pl.pallas_call(
        _tc_layer_body,
        grid=(GRID,),
        in_specs=[
            pl.BlockSpec((RB, fin), lambda i: (i, 0)),
            pl.BlockSpec((fin, C), lambda i: (0, 0)),
            pl.BlockSpec((C, 2), lambda i: (0, 0)),
        ],
        out_specs=[
            pl.BlockSpec((RB, 128), lambda i: (i, 0)),
            pl.BlockSpec((RB, 128), lambda i: (i, 0)),
            pl.BlockSpec((RB, 1), lambda i: (i, 0)),
            pl.BlockSpec((RB, 1), lambda i: (i, 0)),
            pl.BlockSpec((1, 128), lambda i: (0, 0)),
        ],
        out_shape=[
            jax.ShapeDtypeStruct((NP, 128), _F32),
            jax.ShapeDtypeStruct((NP, 128), _F32),
            jax.ShapeDtypeStruct((NP, 1), _F32),
            jax.ShapeDtypeStruct((NP, 1), _F32),
            jax.ShapeDtypeStruct((1, 128), _F32),
        ],
    )(xp, wp, a2)


def _tc_layer12_body(r0_ref, r1_ref, d_ref, b_ref, w_ref, a_ref,
                     h0_ref, h1_ref, as_ref, ad_ref, m_ref):
    i = pl.program_id(0)
    xr = jnp.concatenate([r0_ref[...], r1_ref[...]], axis=1)
    xr = xr / jnp.maximum(d_ref[...], 1e-16) + b_ref[...]
    xr = jnp.maximum(xr, 0.0)
    h = _dot(xr, w_ref[...])
    h0_ref[...] = h[:, :128]
    h1_ref[...] = h[:, 128:]
    aa = _dot(h, a_ref[...])
    as_ref[...] = aa[:, 0:1]
    ad_ref[...] = aa[:, 1:2]
    m2 = jnp.max(aa, axis=0, keepdims=True)
    m2p = jnp.pad(m2, ((0, 0), (0, 126)), constant_values=-1e30)

    @pl.when(i == 0)
    def _():
        m_ref[...] = jnp.full((1, 128), -1e30, _F32)

    m_ref[...] = jnp.maximum(m_ref[...], m2p)


def _tc_layer12(raw0, raw1, den2, b2, w, a2):
    return pl.pallas_call(
        _tc_layer12_body,
        grid=(GRID,),
        in_specs=[
            pl.BlockSpec((RB, 128), lambda i: (i, 0)),
            pl.BlockSpec((RB, 128), lambda i: (i, 0)),
            pl.BlockSpec((RB, 1), lambda i: (i, 0)),
            pl.BlockSpec((1, C), lambda i: (0, 0)),
            pl.BlockSpec((C, C), lambda i: (0, 0)),
            pl.BlockSpec((C, 2), lambda i: (0, 0)),
        ],
        out_specs=[
            pl.BlockSpec((RB, 128), lambda i: (i, 0)),
            pl.BlockSpec((RB, 128), lambda i: (i, 0)),
            pl.BlockSpec((RB, 1), lambda i: (i, 0)),
            pl.BlockSpec((RB, 1), lambda i: (i, 0)),
            pl.BlockSpec((1, 128), lambda i: (0, 0)),
        ],
        out_shape=[
            jax.ShapeDtypeStruct((NP, 128), _F32),
            jax.ShapeDtypeStruct((NP, 128), _F32),
            jax.ShapeDtypeStruct((NP, 1), _F32),
            jax.ShapeDtypeStruct((NP, 1), _F32),
            jax.ShapeDtypeStruct((1, 128), _F32),
        ],
    )(raw0, raw1, den2, b2, w, a2)


def _head_body(r0_ref, r1_ref, d_ref, b_ref, bat_ref, lw_ref, lb_ref,
               o_ref, s_ref, c_ref):
    i = pl.program_id(0)

    @pl.when(i == 0)
    def _():
        s_ref[...] = jnp.zeros((B, C), _F32)
        c_ref[...] = jnp.zeros((B, 128), _F32)

    h3 = jnp.concatenate([r0_ref[...], r1_ref[...]], axis=1)
    h3 = h3 / jnp.maximum(d_ref[...], 1e-16) + b_ref[...]
    bat = bat_ref[0, 0, :]
    rid = lax.broadcasted_iota(jnp.int32, (B, RB), 0)
    oh = (bat[None, :] == rid).astype(_F32)
    s_ref[...] += _dot(oh, h3)
    cnt = jnp.sum(oh, axis=1, keepdims=True)
    c_ref[...] += jnp.broadcast_to(cnt, (B, 128))

    @pl.when(i == GRID - 1)
    def _():
        pooled = s_ref[...] / jnp.maximum(c_ref[:, 0:1], 1.0)
        val = _dot(pooled, lw_ref[...]) + lb_ref[...]
        o_ref[...] = jax.nn.sigmoid(val)


def _head(raw0, raw1, den2, b2, bat3, lw, lb2):
    return pl.pallas_call(
        _head_body,
        grid=(GRID,),
        in_specs=[
            pl.BlockSpec((RB, 128), lambda i: (i, 0)),
            pl.BlockSpec((RB, 128), lambda i: (i, 0)),
            pl.BlockSpec((RB, 1), lambda i: (i, 0)),
            pl.BlockSpec((1, C), lambda i: (0, 0)),
            pl.BlockSpec((1, 1, RB), lambda i: (i, 0, 0)),
            pl.BlockSpec((C, 1), lambda i: (0, 0)),
            pl.BlockSpec((1, 1), lambda i: (0, 0)),
        ],
        out_specs=pl.BlockSpec((B, 1), lambda i: (0, 0)),
        out_shape=jax.ShapeDtypeStruct((B, 1), _F32),
        scratch_shapes=[
            pltpu.VMEM((B, C), _F32),
            pltpu.VMEM((B, 128), _F32),
        ],
    )(raw0, raw1, den2, b2, bat3, lw, lb2)


_MESH = plsc.VectorSubcoreMesh(core_axis_name="c", subcore_axis_name="s")

_SC_CP = pltpu.CompilerParams()
if "needs_layout_passes" in pltpu.CompilerParams.__dataclass_fields__:
    import dataclasses as _dc
    _SC_CP = _dc.replace(_SC_CP, needs_layout_passes=False)

NBLK = CH // 8


@functools.partial(
    pl.kernel,
    mesh=_MESH,
    compiler_params=_SC_CP,
    out_type=[
        jax.ShapeDtypeStruct((NSUB, CH, K), _F32),
        jax.ShapeDtypeStruct((NP,), _F32),
    ],
    scratch_types=[
        pltpu.VMEM((NP,), _F32),
        pltpu.VMEM((NP,), _F32),
        pltpu.VMEM((CH, K), jnp.int32),
        pltpu.VMEM((CH, K), jnp.int32),
        pltpu.VMEM((CH, K), _F32),
        pltpu.VMEM((2, K), _F32),
        pltpu.VMEM((16,), _F32),
        pltpu.VMEM_SHARED((NP,), _F32),
        pltpu.SemaphoreType.DMA((2,)),
    ],
)
def _sc_ee(srcm_h, dstm_h, as_h, ad_h, m_h, zd_h, eem_h, den_h,
           asb, adb, srcb, dstb, eeball, eeb, mb, denacc, dsem):
    c = lax.axis_index("c")
    s = lax.axis_index("s")
    nb = s * NPS

    @pl.when(c == 0)
    def _():
        pltpu.sync_copy(as_h, asb)
        pltpu.sync_copy(ad_h, adb)
        pltpu.sync_copy(m_h, mb)
        pltpu.sync_copy(srcm_h.at[s], srcb)
        pltpu.sync_copy(dstm_h.at[s], dstb)
        pltpu.sync_copy(zd_h, denacc.at[pl.ds(nb, NPS)])
        plsc.subcore_barrier()

        mv = mb[...]
        ebase = s * (CH * K)
        iota16 = lax.broadcasted_iota(jnp.int32, (16,), 0)

        def wait_den(bb, idx):
            pltpu.make_async_copy(eeb.at[bb], denacc.at[idx], dsem.at[bb]).wait()

        @pl.loop(0, CH // 2)
        def _(ii):
            for b in range(2):
                ci = ii * 2 + b
                idx_d = dstb.at[ci]
                pl.when(ii > 0)(lambda: wait_den(b, idx_d))
                for g in range(K // 16):
                    sv = srcb[ci, pl.ds(g * 16, 16)]
                    dv = dstb[ci, pl.ds(g * 16, 16)]
                    a1 = plsc.load_gather(asb, [sv])
                    a2 = plsc.load_gather(adb, [dv])
                    z = a1 + a2
                    e = jnp.maximum(z, 0.2 * z)
                    ee = jnp.exp(e - mv)
                    gid = ebase + ci * K + g * 16 + iota16
                    ee = jnp.where(gid < ETOT, ee, 0.0)
                    eeball[ci, pl.ds(g * 16, 16)] = ee
                    eeb[b, pl.ds(g * 16, 16)] = ee
                pltpu.async_copy(eeb.at[b], denacc.at[idx_d],
                                 dsem.at[b], add=True)

        wait_den(0, dstb.at[0])
        wait_den(1, dstb.at[1])
        pltpu.sync_copy(eeball, eem_h.at[s])
        plsc.subcore_barrier()
        pltpu.sync_copy(denacc.at[pl.ds(nb, NPS)], den_h.at[pl.ds(nb, NPS)])


@functools.partial(
    pl.kernel,
    mesh=_MESH,
    compiler_params=_SC_CP,
    out_type=[
        jax.ShapeDtypeStruct((NP, 128), _F32),
        jax.ShapeDtypeStruct((NP, 128), _F32),
    ],
    scratch_types=[
        pltpu.VMEM((2, 8, K), jnp.int32),
        pltpu.VMEM((2, 8, K), jnp.int32),
        pltpu.VMEM((2, 8, K), _F32),
        pltpu.VMEM((2, K, 128), _F32),
        pltpu.VMEM_SHARED((NP, 128), _F32),
        pltpu.SemaphoreType.DMA((2,)),
        pltpu.SemaphoreType.DMA((2,)),
        pltpu.SemaphoreType.DMA((2,)),
    ],
)
def _sc_agg(srcm_h, dstm_h, eem_h, hh0_h, hh1_h, zr_h, raw0_h, raw1_h,
            srcst, dstst, eest, rows, accum, grsem, ssem, stsem):
    c = lax.axis_index("c")
    s = lax.axis_index("s")
    nb = s * NPS

    pltpu.sync_copy(zr_h, accum.at[pl.ds(nb, NPS)])
    pltpu.sync_copy(srcm_h.at[s, pl.ds(0, 8)], srcst.at[0])
    pltpu.sync_copy(dstm_h.at[s, pl.ds(0, 8)], dstst.at[0])
    pltpu.sync_copy(eem_h.at[s, pl.ds(0, 8)], eest.at[0])
    plsc.subcore_barrier()

    z16 = jnp.zeros((16,), jnp.int32)

    def _half(hh_h):
        def g_issue(bb, pp, jj):
            for h in range(2):
                pltpu.async_copy(
                    hh_h.at[srcst.at[pp, jj, pl.ds(h * 64, 64)]],
                    rows.at[bb, pl.ds(h * 64, 64)], grsem.at[bb])

        def g_wait(bb, idx):
            for h in range(2):
                pltpu.make_async_copy(
                    hh_h.at[idx.at[pl.ds(h * 64, 64)]],
                    rows.at[bb, pl.ds(h * 64, 64)],
                    grsem.at[bb]).wait()

        def s_wait(bb, idx):
            pltpu.make_async_copy(rows.at[bb], accum.at[idx],
                                  ssem.at[bb]).wait()

        g_issue(0, 0, 0)

        @pl.loop(0, NBLK)
        def _(blk):
            p = blk % 2
            for j in range(8):
                b = j % 2
                idx_s = srcst.at[p, j]
                idx_d = dstst.at[p, j]

                g_wait(b, idx_s)

                if j == 0:
                    pl.when(blk > 0)(lambda: s_wait(1 - b, idx_d))
                    g_issue(1, p, 1)
                elif j < 7:
                    s_wait(1 - b, idx_d)
                    g_issue(1 - b, p, j + 1)
                else:
                    def tail():
                        s_wait(1 - b, idx_d)
                        for sth, stv in ((srcm_h, srcst), (dstm_h, dstst),
                                         (eem_h, eest)):
                            pltpu.make_async_copy(
                                sth.at[s, pl.ds((blk + 1) * 8, 8)],
                                stv.at[1 - p], stsem.at[1 - p]).wait()
                        g_issue(1 - b, 1 - p, 0)
                    pl.when(blk < NBLK - 1)(tail)

                if j == 2:
                    def refill():
                        for sth, stv in ((srcm_h, srcst), (dstm_h, dstst),
                                         (eem_h, eest)):
                            pltpu.async_copy(
                                sth.at[s, pl.ds((blk + 1) * 8, 8)],
                                stv.at[1 - p], stsem.at[1 - p])
                    pl.when(blk < NBLK - 1)(refill)

                @plsc.parallel_loop(0, K, unroll=4)
                def _(k):
                    sc16 = plsc.load_gather(
                        eest, [z16 + p, z16 + j, z16 + k])
                    for jj in range(8):
                        sl = pl.ds(jj * 16, 16)
                        rows[b, k, sl] = rows[b, k, sl] * sc16

                pltpu.async_copy(rows.at[b], accum.at[idx_d],
                                 ssem.at[b], add=True)

        s_wait(0, dstst.at[0, 0])
        s_wait(1, dstst.at[0, 0])

    @pl.when(c == 0)
    def _():
        _half(hh0_h)

    @pl.when(c == 1)
    def _():
        _half(hh1_h)

    plsc.subcore_barrier()

    @pl.when(c == 0)
    def _():
        pltpu.sync_copy(accum.at[pl.ds(nb, NPS)], raw0_h.at[pl.ds(nb, NPS)])

    @pl.when(c == 1)
    def _():
        pltpu.sync_copy(accum.at[pl.ds(nb, NPS)], raw1_h.at[pl.ds(nb, NPS)])


def kernel(x, edge_index, batch, W0, att_src0, att_dst0, bias0,
           W1, att_src1, att_dst1, bias1, W2, att_src2, att_dst2, bias2,
           lin_W, lin_b):
    f32 = _F32
    xp = jnp.pad(x, ((0, NP - N), (0, 3)))
    w0p = jnp.pad(W0, ((0, 3), (0, 0)))
    a0 = jnp.stack([att_src0, att_dst0], axis=1)
    a1 = jnp.stack([att_src1, att_dst1], axis=1)
    a2 = jnp.stack([att_src2, att_dst2], axis=1)

    loop = jnp.arange(N, dtype=jnp.int32)
    padi = jnp.arange(EP - ETOT, dtype=jnp.int32) % N
    srcm = jnp.concatenate([edge_index[0], loop, padi]).reshape(NSUB, CH, K)
    dstm = jnp.concatenate([edge_index[1], loop, padi]).reshape(NSUB, CH, K)

    zr = jnp.zeros((NPS, 128), f32)
    zd = jnp.zeros((NPS,), f32)
    batp = jnp.concatenate(
        [batch, jnp.full((NP - N,), B, jnp.int32)]).reshape(GRID, 1, RB)

    def msplat(mx):
        m = mx[0, 0] + mx[0, 1]
        m = jnp.maximum(m, 0.2 * m)
        return jnp.full((16,), m, f32)

    hh0, hh1, asv, adv, mx = _tc_layer0(xp, w0p, a0)
    eem, den = _sc_ee(srcm, dstm, asv.reshape(NP), adv.reshape(NP),
                      msplat(mx), zd)
    raw0, raw1 = _sc_agg(srcm, dstm, eem, hh0, hh1, zr)

    hh0, hh1, asv, adv, mx = _tc_layer12(
        raw0, raw1, den.reshape(NP, 1), bias0.reshape(1, C), W1, a1)
    eem, den = _sc_ee(srcm, dstm, asv.reshape(NP), adv.reshape(NP),
                      msplat(mx), zd)
    raw0, raw1 = _sc_agg(srcm, dstm, eem, hh0, hh1, zr)

    hh0, hh1, asv, adv, mx = _tc_layer12(
        raw0, raw1, den.reshape(NP, 1), bias1.reshape(1, C), W2, a2)
    eem, den = _sc_ee(srcm, dstm, asv.reshape(NP), adv.reshape(NP),
                      msplat(mx), zd)
    raw0, raw1 = _sc_agg(srcm, dstm, eem, hh0, hh1, zr)

    return _head(raw0, raw1, den.reshape(NP, 1), bias2.reshape(1, C),
                 batp, lin_W, lin_b.reshape(1, 1))


# speedup vs baseline: 1.0452x; 1.0281x over previous
import functools

import jax
import jax.numpy as jnp
from jax import lax
from jax.experimental import pallas as pl
from jax.experimental.pallas import tpu as pltpu
from jax.experimental.pallas import tpu_sc as plsc

N = 10000
NP = 10240
C = 256
B = 64
E = 160000
ETOT = E + N
NSUB = 16
K = 128
CH = 88
EP = NSUB * CH * K
NPS = NP // NSUB
RB = 1024
GRID = NP // RB

_F32 = jnp.float32
_HI = lax.Precision.HIGHEST


def _dot(a, b):
    return lax.dot_general(a, b, (((1,), (0,)), ((), ())),
                           precision=_HI, preferred_element_type=_F32)


def _tc_layer_body(x_ref, w_ref, a_ref, h0_ref, h1_ref, as_ref, ad_ref, m_ref):
    i = pl.program_id(0)
    h = _dot(x_ref[...], w_ref[...])
    h0_ref[...] = h[:, :128]
    h1_ref[...] = h[:, 128:]
    aa = _dot(h, a_ref[...])
    as_ref[...] = aa[:, 0:1]
    ad_ref[...] = aa[:, 1:2]
    m2 = jnp.max(aa, axis=0, keepdims=True)
    m2p = jnp.pad(m2, ((0, 0), (0, 126)), constant_values=-1e30)

    @pl.when(i == 0)
    def _():
        m_ref[...] = jnp.full((1, 128), -1e30, _F32)

    m_ref[...] = jnp.maximum(m_ref[...], m2p)


def _tc_layer0(xp, wp, a2):
    fin = xp.shape[1]
    return pl.pallas_call(
        _tc_layer_body,
        grid=(GRID,),
        in_specs=[
            pl.BlockSpec((RB, fin), lambda i: (i, 0)),
            pl.BlockSpec((fin, C), lambda i: (0, 0)),
            pl.BlockSpec((C, 2), lambda i: (0, 0)),
        ],
        out_specs=[
            pl.BlockSpec((RB, 128), lambda i: (i, 0)),
            pl.BlockSpec((RB, 128), lambda i: (i, 0)),
            pl.BlockSpec((RB, 1), lambda i: (i, 0)),
            pl.BlockSpec((RB, 1), lambda i: (i, 0)),
            pl.BlockSpec((1, 128), lambda i: (0, 0)),
        ],
        out_shape=[
            jax.ShapeDtypeStruct((NP, 128), _F32),
            jax.ShapeDtypeStruct((NP, 128), _F32),
            jax.ShapeDtypeStruct((NP, 1), _F32),
            jax.ShapeDtypeStruct((NP, 1), _F32),
            jax.ShapeDtypeStruct((1, 128), _F32),
        ],
    )(xp, wp, a2)


def _tc_layer12_body(r0_ref, r1_ref, d_ref, b_ref, w_ref, a_ref,
                     h0_ref, h1_ref, as_ref, ad_ref, m_ref):
    i = pl.program_id(0)
    xr = jnp.concatenate([r0_ref[...], r1_ref[...]], axis=1)
    xr = xr / jnp.maximum(d_ref[...], 1e-16) + b_ref[...]
    xr = jnp.maximum(xr, 0.0)
    h = _dot(xr, w_ref[...])
    h0_ref[...] = h[:, :128]
    h1_ref[...] = h[:, 128:]
    aa = _dot(h, a_ref[...])
    as_ref[...] = aa[:, 0:1]
    ad_ref[...] = aa[:, 1:2]
    m2 = jnp.max(aa, axis=0, keepdims=True)
    m2p = jnp.pad(m2, ((0, 0), (0, 126)), constant_values=-1e30)

    @pl.when(i == 0)
    def _():
        m_ref[...] = jnp.full((1, 128), -1e30, _F32)

    m_ref[...] = jnp.maximum(m_ref[...], m2p)


def _tc_layer12(raw0, raw1, den2, b2, w, a2):
    return pl.pallas_call(
        _tc_layer12_body,
        grid=(GRID,),
        in_specs=[
            pl.BlockSpec((RB, 128), lambda i: (i, 0)),
            pl.BlockSpec((RB, 128), lambda i: (i, 0)),
            pl.BlockSpec((RB, 1), lambda i: (i, 0)),
            pl.BlockSpec((1, C), lambda i: (0, 0)),
            pl.BlockSpec((C, C), lambda i: (0, 0)),
            pl.BlockSpec((C, 2), lambda i: (0, 0)),
        ],
        out_specs=[
            pl.BlockSpec((RB, 128), lambda i: (i, 0)),
            pl.BlockSpec((RB, 128), lambda i: (i, 0)),
            pl.BlockSpec((RB, 1), lambda i: (i, 0)),
            pl.BlockSpec((RB, 1), lambda i: (i, 0)),
            pl.BlockSpec((1, 128), lambda i: (0, 0)),
        ],
        out_shape=[
            jax.ShapeDtypeStruct((NP, 128), _F32),
            jax.ShapeDtypeStruct((NP, 128), _F32),
            jax.ShapeDtypeStruct((NP, 1), _F32),
            jax.ShapeDtypeStruct((NP, 1), _F32),
            jax.ShapeDtypeStruct((1, 128), _F32),
        ],
    )(raw0, raw1, den2, b2, w, a2)


def _head_body(r0_ref, r1_ref, d_ref, b_ref, bat_ref, lw_ref, lb_ref,
               o_ref, s_ref, c_ref):
    i = pl.program_id(0)

    @pl.when(i == 0)
    def _():
        s_ref[...] = jnp.zeros((B, C), _F32)
        c_ref[...] = jnp.zeros((B, 128), _F32)

    h3 = jnp.concatenate([r0_ref[...], r1_ref[...]], axis=1)
    h3 = h3 / jnp.maximum(d_ref[...], 1e-16) + b_ref[...]
    bat = bat_ref[0, 0, :]
    rid = lax.broadcasted_iota(jnp.int32, (B, RB), 0)
    oh = (bat[None, :] == rid).astype(_F32)
    s_ref[...] += _dot(oh, h3)
    cnt = jnp.sum(oh, axis=1, keepdims=True)
    c_ref[...] += jnp.broadcast_to(cnt, (B, 128))

    @pl.when(i == GRID - 1)
    def _():
        pooled = s_ref[...] / jnp.maximum(c_ref[:, 0:1], 1.0)
        val = _dot(pooled, lw_ref[...]) + lb_ref[...]
        o_ref[...] = jax.nn.sigmoid(val)


def _head(raw0, raw1, den2, b2, bat3, lw, lb2):
    return pl.pallas_call(
        _head_body,
        grid=(GRID,),
        in_specs=[
            pl.BlockSpec((RB, 128), lambda i: (i, 0)),
            pl.BlockSpec((RB, 128), lambda i: (i, 0)),
            pl.BlockSpec((RB, 1), lambda i: (i, 0)),
            pl.BlockSpec((1, C), lambda i: (0, 0)),
            pl.BlockSpec((1, 1, RB), lambda i: (i, 0, 0)),
            pl.BlockSpec((C, 1), lambda i: (0, 0)),
            pl.BlockSpec((1, 1), lambda i: (0, 0)),
        ],
        out_specs=pl.BlockSpec((B, 1), lambda i: (0, 0)),
        out_shape=jax.ShapeDtypeStruct((B, 1), _F32),
        scratch_shapes=[
            pltpu.VMEM((B, C), _F32),
            pltpu.VMEM((B, 128), _F32),
        ],
    )(raw0, raw1, den2, b2, bat3, lw, lb2)


_MESH = plsc.VectorSubcoreMesh(core_axis_name="c", subcore_axis_name="s")

_SC_CP = pltpu.CompilerParams()
if "needs_layout_passes" in pltpu.CompilerParams.__dataclass_fields__:
    import dataclasses as _dc
    _SC_CP = _dc.replace(_SC_CP, needs_layout_passes=False)

NBLK = CH // 8


@functools.partial(
    pl.kernel,
    mesh=_MESH,
    compiler_params=_SC_CP,
    out_type=[
        jax.ShapeDtypeStruct((NP, 128), _F32),
        jax.ShapeDtypeStruct((NP, 128), _F32),
        jax.ShapeDtypeStruct((NP,), _F32),
    ],
    scratch_types=[
        pltpu.VMEM((2, 8, K), jnp.int32),
        pltpu.VMEM((2, 8, K), jnp.int32),
        pltpu.VMEM((CH, K), _F32),
        pltpu.VMEM((2, K), _F32),
        pltpu.VMEM((16,), _F32),
        pltpu.VMEM_SHARED((NP, 128), _F32),
        pltpu.VMEM_SHARED((NP,), _F32),
        pltpu.SemaphoreType.DMA((2,)),
        pltpu.SemaphoreType.DMA((2,)),
        pltpu.SemaphoreType.DMA((2,)),
        pltpu.SemaphoreType.DMA((2,)),
    ],
)
def _sc_layer(srcm_h, dstm_h, hh0_h, hh1_h, as_h, ad_h, m_h, zr_h, zd_h,
              raw0_h, raw1_h, den_h,
              srcst, dstst, eeball, eeb, mb, accum, denacc,
              grsem, ssem, stsem, dsem):
    c = lax.axis_index("c")
    s = lax.axis_index("s")
    nb = s * NPS

    pltpu.sync_copy(zr_h, accum.at[pl.ds(nb, NPS)])
    pltpu.sync_copy(m_h, mb)
    pltpu.sync_copy(srcm_h.at[s, pl.ds(0, 8)], srcst.at[0])
    pltpu.sync_copy(dstm_h.at[s, pl.ds(0, 8)], dstst.at[0])

    @pl.when(c == 0)
    def _():
        pltpu.sync_copy(zd_h, denacc.at[pl.ds(nb, NPS)])

    plsc.subcore_barrier()

    mv = mb[...]
    ebase = s * (CH * K)
    iota16 = lax.broadcasted_iota(jnp.int32, (16,), 0)
    z16 = jnp.zeros((16,), jnp.int32)

    def phase1(asb, adb):
        pltpu.sync_copy(as_h, asb)
        pltpu.sync_copy(ad_h, adb)

        def wait_den(bb, idx):
            pltpu.make_async_copy(eeb.at[bb], denacc.at[idx],
                                  dsem.at[bb]).wait()

        @pl.loop(0, NBLK)
        def _(blk):
            p2 = blk % 2
            for j in range(8):
                b = j % 2
                ci = blk * 8 + j
                idx_d = dstst.at[p2, j]

                @pl.when(c == 0)
                def _():
                    if j >= 2:
                        wait_den(b, idx_d)
                    else:
                        pl.when(blk > 0)(lambda: wait_den(b, idx_d))

                if j == 2:
                    def refill1():
                        for sth, stv in ((srcm_h, srcst), (dstm_h, dstst)):
                            pltpu.async_copy(
                                sth.at[s, pl.ds((blk + 1) * 8, 8)],
                                stv.at[1 - p2], stsem.at[1 - p2])
                    pl.when(blk < NBLK - 1)(refill1)

                for g in range(K // 16):
                    sv = srcst[p2, j, pl.ds(g * 16, 16)]
                    dv = dstst[p2, j, pl.ds(g * 16, 16)]
                    a1 = plsc.load_gather(asb, [sv])
                    a2 = plsc.load_gather(adb, [dv])
                    z = a1 + a2
                    e = jnp.maximum(z, 0.2 * z)
                    ee = jnp.exp(e - mv)
                    gid = ebase + ci * K + g * 16 + iota16
                    ee = jnp.where(gid < ETOT, ee, 0.0)
                    eeball[ci, pl.ds(g * 16, 16)] = ee
                    eeb[b, pl.ds(g * 16, 16)] = ee

                @pl.when(c == 0)
                def _():
                    pltpu.async_copy(eeb.at[b], denacc.at[idx_d],
                                     dsem.at[b], add=True)

                if j == 7:
                    def wrefill1():
                        for sth, stv in ((srcm_h, srcst), (dstm_h, dstst)):
                            pltpu.make_async_copy(
                                sth.at[s, pl.ds((blk + 1) * 8, 8)],
                                stv.at[1 - p2], stsem.at[1 - p2]).wait()
                    pl.when(blk < NBLK - 1)(wrefill1)

        @pl.when(c == 0)
        def _():
            wait_den(0, dstst.at[0, 0])
            wait_den(1, dstst.at[0, 1])

    pl.run_scoped(phase1, pltpu.VMEM((NP,), _F32), pltpu.VMEM((NP,), _F32))

    pltpu.sync_copy(srcm_h.at[s, pl.ds(0, 8)], srcst.at[0])
    pltpu.sync_copy(dstm_h.at[s, pl.ds(0, 8)], dstst.at[0])

    def phase2(rows, hh_h):
        def g_issue(bb, pp, jj):
            for h in range(2):
                pltpu.async_copy(
                    hh_h.at[srcst.at[pp, jj, pl.ds(h * 64, 64)]],
                    rows.at[bb, pl.ds(h * 64, 64)], grsem.at[bb])

        def g_wait(bb, idx):
            for h in range(2):
                pltpu.make_async_copy(
                    hh_h.at[idx.at[pl.ds(h * 64, 64)]],
                    rows.at[bb, pl.ds(h * 64, 64)],
                    grsem.at[bb]).wait()

        def s_wait(bb, idx):
            pltpu.make_async_copy(rows.at[bb], accum.at[idx],
                                  ssem.at[bb]).wait()

        g_issue(0, 0, 0)

        @pl.loop(0, NBLK)
        def _(blk):
            p = blk % 2
            for j in range(8):
                b = j % 2
                ci = blk * 8 + j
                idx_s = srcst.at[p, j]
                idx_d = dstst.at[p, j]

                g_wait(b, idx_s)

                if j == 0:
                    pl.when(blk > 0)(lambda: s_wait(1 - b, idx_d))
                    g_issue(1, p, 1)
                elif j < 7:
                    s_wait(1 - b, idx_d)
                    g_issue(1 - b, p, j + 1)
                else:
                    def tail():
                        s_wait(1 - b, idx_d)
                        for sth, stv in ((srcm_h, srcst), (dstm_h, dstst)):
                            pltpu.make_async_copy(
                                sth.at[s, pl.ds((blk + 1) * 8, 8)],
                                stv.at[1 - p], stsem.at[1 - p]).wait()
                        g_issue(1 - b, 1 - p, 0)
                    pl.when(blk < NBLK - 1)(tail)

                if j == 2:
                    def refill():
                        for sth, stv in ((srcm_h, srcst), (dstm_h, dstst)):
                            pltpu.async_copy(
                                sth.at[s, pl.ds((blk + 1) * 8, 8)],
                                stv.at[1 - p], stsem.at[1 - p])
                    pl.when(blk < NBLK - 1)(refill)

                @plsc.parallel_loop(0, K, unroll=4)
                def _(k):
                    sc16 = plsc.load_gather(eeball, [z16 + ci, z16 + k])
                    for jj in range(8):
                        sl = pl.ds(jj * 16, 16)
                        rows[b, k, sl] = rows[b, k, sl] * sc16

                pltpu.async_copy(rows.at[b], accum.at[idx_d],
                                 ssem.at[b], add=True)

        s_wait(0, dstst.at[0, 0])
        s_wait(1, dstst.at[0, 1])

    @pl.when(c == 0)
    def _():
        pl.run_scoped(functools.partial(phase2, hh_h=hh0_h),
                      pltpu.VMEM((2, K, 128), _F32))

    @pl.when(c == 1)
    def _():
        pl.run_scoped(functools.partial(phase2, hh_h=hh1_h),
                      pltpu.VMEM((2, K, 128), _F32))

    plsc.subcore_barrier()

    @pl.when(c == 0)
    def _():
        pltpu.sync_copy(accum.at[pl.ds(nb, NPS)], raw0_h.at[pl.ds(nb, NPS)])
        pltpu.sync_copy(denacc.at[pl.ds(nb, NPS)], den_h.at[pl.ds(nb, NPS)])

    @pl.when(c == 1)
    def _():
        pltpu.sync_copy(accum.at[pl.ds(nb, NPS)], raw1_h.at[pl.ds(nb, NPS)])


def kernel(x, edge_index, batch, W0, att_src0, att_dst0, bias0,
           W1, att_src1, att_dst1, bias1, W2, att_src2, att_dst2, bias2,
           lin_W, lin_b):
    f32 = _F32
    xp = jnp.pad(x, ((0, NP - N), (0, 3)))
    w0p = jnp.pad(W0, ((0, 3), (0, 0)))
    a0 = jnp.stack([att_src0, att_dst0], axis=1)
    a1 = jnp.stack([att_src1, att_dst1], axis=1)
    a2 = jnp.stack([att_src2, att_dst2], axis=1)

    loop = jnp.arange(N, dtype=jnp.int32)
    padi = jnp.arange(EP - ETOT, dtype=jnp.int32) % N
    srcm = jnp.concatenate([edge_index[0], loop, padi]).reshape(NSUB, CH, K)
    dstm = jnp.concatenate([edge_index[1], loop, padi]).reshape(NSUB, CH, K)

    zr = jnp.zeros((NPS, 128), f32)
    zd = jnp.zeros((NPS,), f32)
    batp = jnp.concatenate(
        [batch, jnp.full((NP - N,), B, jnp.int32)]).reshape(GRID, 1, RB)

    def msplat(mx):
        m = mx[0, 0] + mx[0, 1]
        m = jnp.maximum(m, 0.2 * m)
        return jnp.full((16,), m, f32)

    hh0, hh1, asv, adv, mx = _tc_layer0(xp, w0p, a0)
    raw0, raw1, den = _sc_layer(srcm, dstm, hh0, hh1,
                                asv.reshape(NP), adv.reshape(NP),
                                msplat(mx), zr, zd)

    hh0, hh1, asv, adv, mx = _tc_layer12(
        raw0, raw1, den.reshape(NP, 1), bias0.reshape(1, C), W1, a1)
    raw0, raw1, den = _sc_layer(srcm, dstm, hh0, hh1,
                                asv.reshape(NP), adv.reshape(NP),
                                msplat(mx), zr, zd)

    hh0, hh1, asv, adv, mx = _tc_layer12(
        raw0, raw1, den.reshape(NP, 1), bias1.reshape(1, C), W2, a2)
    raw0, raw1, den = _sc_layer(srcm, dstm, hh0, hh1,
                                asv.reshape(NP), adv.reshape(NP),
                                msplat(mx), zr, zd)

    return _head(raw0, raw1, den.reshape(NP, 1), bias2.reshape(1, C),
                 batp, lin_W, lin_b.reshape(1, 1))
